# Initial kernel scaffold; baseline (speedup 1.0000x reference)
import jax
import jax.numpy as jnp
from jax.experimental import pallas as pl


def kernel(edge_features, edge_coords, edge_index, node_coords, block_ids, Wq, Wk, Wv, Wo, Wrbf, brbf, Wg, W1, b1, W2, b2, ln1_s, ln1_b, ln2_s, ln2_b):
    raise NotImplementedError("write your pallas kernel here")


# speedup vs baseline: 3.6986x; 3.6986x over previous
import jax
import jax.numpy as jnp
from jax.experimental import pallas as pl

E = 2048
N = 512
D = 128
H = 8
DH = D // H
L = 2
R = 64
CUT = 10.0
FF = 4 * D

TQ = 512
NT = E // TQ

_NEG = -1e9


def _ln(x, s, b):
    m = jnp.mean(x, axis=-1, keepdims=True)
    v = jnp.mean((x - m) ** 2, axis=-1, keepdims=True)
    return (x - m) / jnp.sqrt(v + 1e-5) * s + b


def _body(src_ref, dst_ref, bidr_ref, bidc_ref, ncp_ref, feat_ref, c0_ref,
          wq_ref, wk_ref, wv_ref, wo_ref, wrbf_ref, brbf_ref, wg_ref,
          w1_ref, b1_ref, w2_ref, b2_ref, l1s_ref, l1b_ref, l2s_ref, l2b_ref,
          fout_ref, cout_ref):
    f32 = jnp.float32
    hi = jax.lax.Precision.HIGHEST

    iota_n = jax.lax.broadcasted_iota(jnp.int32, (E, N), 1)
    oh_s = (src_ref[...] == iota_n).astype(f32)
    oh_d = (dst_ref[...] == iota_n).astype(f32)

    eb_col = jax.lax.dot_general(oh_s, bidc_ref[...],
                                 (((1,), (0,)), ((), ())), precision=hi)
    eb_row = jax.lax.dot_general(bidr_ref[...], oh_s,
                                 (((1,), (1,)), ((), ())), precision=hi)

    ncp = ncp_ref[...]
    cs = jax.lax.dot_general(oh_s, ncp, (((1,), (0,)), ((), ())), precision=hi)
    cd = jax.lax.dot_general(oh_d, ncp, (((1,), (0,)), ((), ())), precision=hi)
    diff = cs - cd
    d_col = jnp.sqrt(jnp.sum(diff * diff, axis=1, keepdims=True) + 1e-12)

    centers = jax.lax.broadcasted_iota(jnp.int32, (1, R), 1).astype(f32) * (CUT / (R - 1))
    env = 0.5 * (jnp.cos(jnp.pi * jnp.clip(d_col, 0.0, CUT) / CUT) + 1.0)
    rbf = jnp.exp(-(10.0 / CUT) * (d_col - centers) ** 2) * env

    wq = wq_ref[...]
    wk = wk_ref[...]
    wv = wv_ref[...]
    wo = wo_ref[...]
    wrbf = wrbf_ref[...]
    brbf = brbf_ref[...]
    wg = wg_ref[...]
    w1 = w1_ref[...]
    b1 = b1_ref[...]
    w2 = w2_ref[...]
    b2 = b2_ref[...]
    l1s = l1s_ref[...]
    l1b = l1b_ref[...]
    l2s = l2s_ref[...]
    l2b = l2b_ref[...]

    f = feat_ref[...]
    c = c0_ref[...]

    ones_col = jnp.ones((TQ, 1), f32)

    for l in range(L):
        h = _ln(f, l1s[l], l1b[l])
        q = jnp.dot(h, wq[l]) * (1.0 / jnp.sqrt(f32(DH)))
        k = jnp.dot(h, wk[l])
        v = jnp.dot(h, wv[l])
        bias = jnp.dot(rbf, wrbf[l]) + brbf[l]

        attn_rows = []
        cdel_rows = []
        for t in range(NT):
            qs = slice(t * TQ, (t + 1) * TQ)
            mask = eb_col[qs] == eb_row
            ai_sum = jnp.zeros((TQ, E), f32)
            outs = []
            for hh in range(H):
                hsl = slice(hh * DH, (hh + 1) * DH)
                qa = jnp.concatenate([q[qs, hsl], ones_col], axis=1)
                ka = jnp.concatenate([k[:, hsl], bias[:, hh:hh + 1]], axis=1)
                lg = jax.lax.dot_general(qa, ka, (((1,), (1,)), ((), ())))
                li = jnp.where(mask, lg, _NEG)
                ei = jnp.exp(li - jnp.max(li, axis=1, keepdims=True))
                ai = ei / jnp.sum(ei, axis=1, keepdims=True)
                lj = jnp.where(mask, _NEG, lg)
                ej = jnp.exp(lj - jnp.max(lj, axis=1, keepdims=True))
                aj = ej / jnp.sum(ej, axis=1, keepdims=True)
                outs.append(jnp.dot(ai + aj, v[:, hsl]))
                ai_sum = ai_sum + ai
            attn_rows.append(jnp.concatenate(outs, axis=1))
            cdel_rows.append(jnp.dot(ai_sum, c) * (1.0 / H))

        attn = jnp.concatenate(attn_rows, axis=0)
        cdel = jnp.concatenate(cdel_rows, axis=0)
        f = f + jnp.dot(attn, wo[l])
        gate = jnp.tanh(jnp.sum(h * wg[l], axis=1, keepdims=True))
        c = c + gate * cdel
        h2 = _ln(f, l2s[l], l2b[l])
        u = jnp.dot(h2, w1[l]) + b1[l]
        u = u * jax.nn.sigmoid(u)
        f = f + jnp.dot(u, w2[l]) + b2[l]

    fout_ref[...] = f
    cout_ref[...] = c


def kernel(edge_features, edge_coords, edge_index, node_coords, block_ids,
           Wq, Wk, Wv, Wo, Wrbf, brbf, Wg, W1, b1, W2, b2,
           ln1_s, ln1_b, ln2_s, ln2_b):
    f32 = jnp.float32
    src = edge_index[0].astype(jnp.int32).reshape(E, 1)
    dst = edge_index[1].astype(jnp.int32).reshape(E, 1)
    bidr = block_ids.astype(f32).reshape(1, N)
    bidc = block_ids.astype(f32).reshape(N, 1)
    ncp = jnp.pad(node_coords.astype(f32), ((0, 0), (0, 5)))
    c0 = jnp.pad(edge_coords.astype(f32), ((0, 0), (0, 5)))

    f_out, c_out = pl.pallas_call(
        _body,
        out_shape=[
            jax.ShapeDtypeStruct((E, D), f32),
            jax.ShapeDtypeStruct((E, 8), f32),
        ],
    )(src, dst, bidr, bidc, ncp, edge_features, c0,
      Wq, Wk, Wv, Wo, Wrbf, brbf.reshape(L, 1, H), Wg.reshape(L, 1, D),
      W1, b1.reshape(L, 1, FF), W2, b2.reshape(L, 1, D),
      ln1_s.reshape(L, 1, D), ln1_b.reshape(L, 1, D),
      ln2_s.reshape(L, 1, D), ln2_b.reshape(L, 1, D))
    return f_out, c_out[:, :3]
